# Initial kernel scaffold; baseline (speedup 1.0000x reference)
import jax
import jax.numpy as jnp
from jax.experimental import pallas as pl


def kernel(input_ids, table, ti_vec, out_dtype):
    raise NotImplementedError("write your pallas kernel here")


# speedup vs baseline: 3.5839x; 3.5839x over previous
import functools

import jax
import jax.numpy as jnp
from jax import lax
from jax.experimental import pallas as pl
from jax.experimental.pallas import tpu as pltpu
from jax.experimental.pallas import tpu_sc as plsc

VOCAB = 100000
D = 128
B = 4096
S = 50
TI_LEN = 8
TI_START = 11

NC, NS = 2, 16
NW = NC * NS
N_ROWS = B * S
ROWS_PER_W = N_ROWS // NW
SEQS_PER_CHUNK = 8
CHUNK = SEQS_PER_CHUNK * S
N_CHUNKS = ROWS_PER_W // CHUNK

@functools.cache
def _build_sc_embed():
    mesh = plsc.VectorSubcoreMesh(
        core_axis_name="c", subcore_axis_name="s", num_cores=NC, num_subcores=NS
    )

    @functools.partial(
        pl.kernel,
        mesh=mesh,
        out_type=jax.ShapeDtypeStruct((N_ROWS, D), jnp.float32),
        scratch_types=[
            pltpu.VMEM((CHUNK,), jnp.int32),
            pltpu.VMEM((CHUNK, D), jnp.float32),
            pltpu.VMEM((TI_LEN, D), jnp.float32),
            pltpu.SemaphoreType.DMA,
        ],
    )
    def _sc_embed(ids_hbm, table_hbm, ti_hbm, out_hbm, idx_v, buf, ti_v, sem):
        wid = lax.axis_index("s") * NC + lax.axis_index("c")
        base = wid * ROWS_PER_W
        pltpu.sync_copy(ti_hbm, ti_v)

        def chunk_body(c, carry):
            row0 = base + c * CHUNK
            pltpu.sync_copy(ids_hbm.at[pl.ds(row0, CHUNK)], idx_v)
            pltpu.async_copy(table_hbm.at[idx_v], buf, sem).wait()
            for r in range(TI_LEN):
                for c8 in range(D // 16):
                    v = ti_v[r, pl.ds(c8 * 16, 16)]
                    for q in range(SEQS_PER_CHUNK):
                        buf[q * S + TI_START + r, pl.ds(c8 * 16, 16)] = v
            pltpu.sync_copy(buf, out_hbm.at[pl.ds(row0, CHUNK)])
            return carry

        lax.fori_loop(0, N_CHUNKS, chunk_body, 0)

    return _sc_embed


def kernel(input_ids, table, ti_vec, out_dtype):
    del out_dtype
    ids_flat = input_ids.reshape(N_ROWS).astype(jnp.int32)
    out = _build_sc_embed()(ids_flat, table, ti_vec)
    return out.reshape(B, S, D)


# speedup vs baseline: 3.7927x; 1.0583x over previous
import functools

import jax
import jax.numpy as jnp
from jax import lax
from jax.experimental import pallas as pl
from jax.experimental.pallas import tpu as pltpu
from jax.experimental.pallas import tpu_sc as plsc

VOCAB = 100000
D = 128
B = 4096
S = 50
TI_LEN = 8
TI_START = 11

NC, NS = 2, 16
NW = NC * NS
N_ROWS = B * S
ROWS_PER_W = N_ROWS // NW
SEQS_PER_CHUNK = 8
CHUNK = SEQS_PER_CHUNK * S
N_CHUNKS = ROWS_PER_W // CHUNK

@functools.cache
def _build_sc_embed():
    mesh = plsc.VectorSubcoreMesh(
        core_axis_name="c", subcore_axis_name="s", num_cores=NC, num_subcores=NS
    )

    @functools.partial(
        pl.kernel,
        mesh=mesh,
        out_type=jax.ShapeDtypeStruct((N_ROWS, D), jnp.float32),
        scratch_types=[
            pltpu.VMEM((ROWS_PER_W,), jnp.int32),
            pltpu.VMEM((TI_LEN, D), jnp.float32),
            pltpu.VMEM((CHUNK, D), jnp.float32),
            pltpu.VMEM((CHUNK, D), jnp.float32),
            pltpu.SemaphoreType.DMA,
            pltpu.SemaphoreType.DMA,
            pltpu.SemaphoreType.DMA,
            pltpu.SemaphoreType.DMA,
        ],
    )
    def _sc_embed(
        ids_hbm, table_hbm, ti_hbm, out_hbm, idx_v, ti_v, buf0, buf1, g0, g1, w0, w1
    ):
        wid = lax.axis_index("s") * NC + lax.axis_index("c")
        base = wid * ROWS_PER_W
        bufs, gsems, wsems = (buf0, buf1), (g0, g1), (w0, w1)
        pltpu.sync_copy(ids_hbm.at[pl.ds(base, ROWS_PER_W)], idx_v)
        pltpu.sync_copy(ti_hbm, ti_v)

        def gather(c, b):
            pltpu.async_copy(
                table_hbm.at[idx_v.at[pl.ds(c * CHUNK, CHUNK)]], bufs[b], gsems[b]
            )

        def gather_wait(c, b):
            pltpu.make_async_copy(
                table_hbm.at[idx_v.at[pl.ds(c * CHUNK, CHUNK)]], bufs[b], gsems[b]
            ).wait()

        gather(0, 0)
        gather(1, 1)

        def pair_body(p, carry):
            for b in range(2):
                c = 2 * p + b
                gather_wait(c, b)
                for r in range(TI_LEN):
                    for c8 in range(D // 16):
                        v = ti_v[r, pl.ds(c8 * 16, 16)]
                        for q in range(SEQS_PER_CHUNK):
                            bufs[b][q * S + TI_START + r, pl.ds(c8 * 16, 16)] = v
                wd = pltpu.async_copy(
                    bufs[b], out_hbm.at[pl.ds(base + c * CHUNK, CHUNK)], wsems[b]
                )
                wd.wait()

                @pl.when(c + 2 < N_CHUNKS)
                def _():
                    gather(c + 2, b)

            return carry

        lax.fori_loop(0, N_CHUNKS // 2, pair_body, 0)

    return _sc_embed


def kernel(input_ids, table, ti_vec, out_dtype):
    del out_dtype
    ids_flat = input_ids.reshape(N_ROWS).astype(jnp.int32)
    out = _build_sc_embed()(ids_flat, table, ti_vec)
    return out.reshape(B, S, D)


# speedup vs baseline: 6.6865x; 1.7630x over previous
import functools

import jax
import jax.numpy as jnp
from jax import lax
from jax.experimental import pallas as pl
from jax.experimental.pallas import tpu as pltpu
from jax.experimental.pallas import tpu_sc as plsc

VOCAB = 100000
D = 128
B = 4096
S = 50
TI_LEN = 8
TI_START = 11

NC, NS = 2, 16
NW = NC * NS
BATCH_PER_W = B // NW
SEQS_PER_CHUNK = 8
N_CHUNKS = BATCH_PER_W // SEQS_PER_CHUNK


@functools.cache
def _build_sc_embed():
    mesh = plsc.VectorSubcoreMesh(
        core_axis_name="c", subcore_axis_name="s", num_cores=NC, num_subcores=NS
    )

    @functools.partial(
        pl.kernel,
        mesh=mesh,
        out_type=jax.ShapeDtypeStruct((B, S, D), jnp.float32),
        scratch_types=[
            pltpu.VMEM((BATCH_PER_W, S), jnp.int32),
            pltpu.VMEM((TI_LEN, D), jnp.float32),
            pltpu.VMEM((SEQS_PER_CHUNK * S, D), jnp.float32),
            pltpu.VMEM((SEQS_PER_CHUNK * S, D), jnp.float32),
            pltpu.SemaphoreType.DMA,
            pltpu.SemaphoreType.DMA,
            pltpu.SemaphoreType.DMA,
            pltpu.SemaphoreType.DMA,
        ],
    )
    def _sc_embed(
        ids_hbm, table_hbm, ti_hbm, out_hbm, idx_v, ti_v, buf0, buf1, g0, g1, w0, w1
    ):
        wid = lax.axis_index("s") * NC + lax.axis_index("c")
        base_b = wid * BATCH_PER_W
        bufs, gsems, wsems = (buf0, buf1), (g0, g1), (w0, w1)
        pltpu.sync_copy(ids_hbm.at[pl.ds(base_b, BATCH_PER_W)], idx_v)
        pltpu.sync_copy(ti_hbm, ti_v)

        def gather(c, b):
            for q in range(SEQS_PER_CHUNK):
                pltpu.async_copy(
                    table_hbm.at[idx_v.at[c * SEQS_PER_CHUNK + q]],
                    bufs[b].at[pl.ds(q * S, S)],
                    gsems[b],
                )

        def gather_wait(c, b):
            for q in range(SEQS_PER_CHUNK):
                pltpu.make_async_copy(
                    table_hbm.at[idx_v.at[c * SEQS_PER_CHUNK + q]],
                    bufs[b].at[pl.ds(q * S, S)],
                    gsems[b],
                ).wait()

        gather(0, 0)
        gather(1, 1)

        def pair_body(p, carry):
            for b in range(2):
                c = 2 * p + b
                gather_wait(c, b)
                for r in range(TI_LEN):
                    for c8 in range(D // 16):
                        v = ti_v[r, pl.ds(c8 * 16, 16)]
                        for q in range(SEQS_PER_CHUNK):
                            bufs[b][q * S + TI_START + r, pl.ds(c8 * 16, 16)] = v
                wds = [
                    pltpu.async_copy(
                        bufs[b].at[pl.ds(q * S, S)],
                        out_hbm.at[base_b + c * SEQS_PER_CHUNK + q],
                        wsems[b],
                    )
                    for q in range(SEQS_PER_CHUNK)
                ]
                for wd in wds:
                    wd.wait()

                @pl.when(c + 2 < N_CHUNKS)
                def _():
                    gather(c + 2, b)

            return carry

        lax.fori_loop(0, N_CHUNKS // 2, pair_body, 0)

    return _sc_embed


def kernel(input_ids, table, ti_vec, out_dtype):
    del out_dtype
    return _build_sc_embed()(input_ids.astype(jnp.int32), table, ti_vec)


# speedup vs baseline: 12.7291x; 1.9037x over previous
import functools

import jax
import jax.numpy as jnp
from jax import lax
from jax.experimental import pallas as pl
from jax.experimental.pallas import tpu as pltpu
from jax.experimental.pallas import tpu_sc as plsc

VOCAB = 100000
D = 128
B = 4096
S = 50
TI_LEN = 8
TI_START = 11

NC, NS = 2, 16
NW = NC * NS
N_ROWS = S * B

TI_LO = TI_START * B
TI_HI = (TI_START + TI_LEN) * B

A_PER_W = TI_LO // NW
B_PER_W = (N_ROWS - TI_HI) // NW
G_PER_W = A_PER_W + B_PER_W
TI_PER_W = (TI_HI - TI_LO) // NW

CHUNK = 384
N_CHUNKS = G_PER_W // CHUNK
FILL_ROWS = 64
N_FILL = TI_PER_W // FILL_ROWS


@functools.cache
def _build_sc_embed():
    mesh = plsc.VectorSubcoreMesh(
        core_axis_name="c", subcore_axis_name="s", num_cores=NC, num_subcores=NS
    )

    @functools.partial(
        pl.kernel,
        mesh=mesh,
        out_type=jax.ShapeDtypeStruct((N_ROWS, D), jnp.float32),
        scratch_types=[
            pltpu.VMEM((G_PER_W,), jnp.int32),
            pltpu.VMEM((FILL_ROWS, D), jnp.float32),
            pltpu.VMEM((1, D), jnp.float32),
            pltpu.VMEM((CHUNK, D), jnp.float32),
            pltpu.VMEM((CHUNK, D), jnp.float32),
            pltpu.SemaphoreType.DMA,
            pltpu.SemaphoreType.DMA,
            pltpu.SemaphoreType.DMA,
            pltpu.SemaphoreType.DMA,
            pltpu.SemaphoreType.DMA,
        ],
    )
    def _sc_embed(
        ids_hbm, table_hbm, ti_hbm, out_hbm,
        idx_v, fill_v, tirow_v, buf0, buf1, g0, g1, w0, w1, tsem,
    ):
        wid = lax.axis_index("s") * NC + lax.axis_index("c")
        bufs, gsems, wsems = (buf0, buf1), (g0, g1), (w0, w1)
        a0 = wid * A_PER_W
        b0 = TI_HI + wid * B_PER_W

        pltpu.sync_copy(ids_hbm.at[pl.ds(a0, A_PER_W)], idx_v.at[pl.ds(0, A_PER_W)])
        pltpu.sync_copy(
            ids_hbm.at[pl.ds(b0, B_PER_W)], idx_v.at[pl.ds(A_PER_W, B_PER_W)]
        )

        s_off = wid // (NW // TI_LEN)
        pltpu.sync_copy(ti_hbm.at[s_off], tirow_v.at[0])
        for c8 in range(D // 16):
            v = tirow_v[0, pl.ds(c8 * 16, 16)]
            for r in range(FILL_ROWS):
                fill_v[r, pl.ds(c8 * 16, 16)] = v
        ti0 = TI_LO + wid * TI_PER_W
        tds = [
            pltpu.async_copy(
                fill_v, out_hbm.at[pl.ds(ti0 + j * FILL_ROWS, FILL_ROWS)], tsem
            )
            for j in range(N_FILL)
        ]

        def out_off(vrow):
            return a0 + vrow if vrow < A_PER_W else b0 + (vrow - A_PER_W)

        def gather(k):
            return pltpu.async_copy(
                table_hbm.at[idx_v.at[pl.ds(k * CHUNK, CHUNK)]],
                bufs[k % 2],
                gsems[k % 2],
            )

        def write(k):
            v0 = k * CHUNK
            wds = []
            if v0 < A_PER_W < v0 + CHUNK:
                la = A_PER_W - v0
                wds.append(
                    pltpu.async_copy(
                        bufs[k % 2].at[pl.ds(0, la)],
                        out_hbm.at[pl.ds(a0 + v0, la)],
                        wsems[k % 2],
                    )
                )
                wds.append(
                    pltpu.async_copy(
                        bufs[k % 2].at[pl.ds(la, CHUNK - la)],
                        out_hbm.at[pl.ds(b0, CHUNK - la)],
                        wsems[k % 2],
                    )
                )
            else:
                wds.append(
                    pltpu.async_copy(
                        bufs[k % 2],
                        out_hbm.at[pl.ds(out_off(v0), CHUNK)],
                        wsems[k % 2],
                    )
                )
            return wds

        gds = {0: gather(0), 1: gather(1)}
        pending_w = {}
        for k in range(N_CHUNKS):
            gds.pop(k).wait()
            pending_w[k] = write(k)
            if k + 2 < N_CHUNKS:
                for wd in pending_w.pop(k):
                    wd.wait()
                gds[k + 2] = gather(k + 2)
        for k, wds in sorted(pending_w.items()):
            for wd in wds:
                wd.wait()
        for td in tds:
            td.wait()

    return _sc_embed


def kernel(input_ids, table, ti_vec, out_dtype):
    del out_dtype
    ids_t = input_ids.astype(jnp.int32).T.reshape(N_ROWS)
    out_flat = _build_sc_embed()(ids_t, table, ti_vec)
    return jnp.transpose(out_flat.reshape(S, B, D), (1, 0, 2))
